# baseline (speedup 1.0000x reference)
import functools

import jax
import jax.numpy as jnp
from jax import lax
from jax.experimental import pallas as pl
from jax.experimental.pallas import tpu as pltpu

_C1 = 192
_CK = 576
_GM = 192

_PMAP = ({-1: 0, 0: 2, 1: 4}, {0: 1, 1: 3})

_DTAPS = [
    (2 * a + b, sy, sx, _PMAP[a][sy], _PMAP[b][sx])
    for a in (0, 1) for b in (0, 1)
    for sy in (-1, 0, 1) if sy in _PMAP[a]
    for sx in (-1, 0, 1) if sx in _PMAP[b]
]

_CTAPS = [(dy, dx) for dy in range(5) for dx in range(5)]


def _build_deconv_taps(w):
    return jnp.stack([w[:, :, 4 - p, 4 - q] for (_, _, _, p, q) in _DTAPS])


def _build_conv_taps(w):
    return jnp.stack([w[:, :, dy, dx] for (dy, dx) in _CTAPS])


def _act(x, leaky):
    return jnp.where(x >= 0, x, 0.01 * x) if leaky else jnp.maximum(x, 0.0)


def _deconv_planes(xs, w_ref, b_ref, leaky, h):
    cache = {}

    def get(sy, sx):
        if (sy, sx) not in cache:
            cache[(sy, sx)] = xs(sy, sx)
        return cache[(sy, sx)]

    planes = []
    for p in range(4):
        acc = None
        for t, (tp, sy, sx, _, _) in enumerate(_DTAPS):
            if tp != p:
                continue
            d = jnp.dot(get(sy, sx), w_ref[t],
                        preferred_element_type=jnp.float32)
            acc = d if acc is None else acc + d
        acc = _act(acc + b_ref[...], leaky)
        planes.append(acc.reshape(h, h, -1))
    row_e = jnp.stack([planes[0], planes[1]], axis=2).reshape(h, 2 * h, -1)
    row_o = jnp.stack([planes[2], planes[3]], axis=2).reshape(h, 2 * h, -1)
    return jnp.stack([row_e, row_o], axis=1).reshape(2 * h, 2 * h, -1)


def _branch_kernel(z1_ref, w1_ref, b1_ref, w2_ref, b2_ref, w3_ref, b3_ref,
                   out_ref, l1pad, l2pad, *, leaky, final_relu):
    def xs1(sy, sx):
        return z1_ref[0, 1 + sy:17 + sy, 1 + sx:17 + sx, :].reshape(256, _C1)

    y1 = _deconv_planes(xs1, w1_ref, b1_ref, leaky, 16)
    l1pad[...] = jnp.zeros_like(l1pad)
    l1pad[1:33, 1:33, :] = y1.astype(jnp.bfloat16)

    def xs2(sy, sx):
        return l1pad[1 + sy:33 + sy, 1 + sx:33 + sx, :].reshape(1024, _C1)

    y2 = _deconv_planes(xs2, w2_ref, b2_ref, leaky, 32)
    l2pad[...] = jnp.zeros_like(l2pad)
    l2pad[2:66, 2:66, :] = y2.astype(jnp.bfloat16)

    acc = None
    for t, (dy, dx) in enumerate(_CTAPS):
        xv = l2pad[dy:dy + 64, dx:dx + 64, :].reshape(4096, _C1)
        d = lax.dot_general(w3_ref[t], xv, (((1,), (1,)), ((), ())),
                            preferred_element_type=jnp.float32)
        acc = d if acc is None else acc + d
    acc = acc + b3_ref[...]
    if final_relu:
        acc = jnp.maximum(acc, 0.0)
    out_ref[0] = acc


def _gw_kernel(z1_ref, w1_ref, b1_ref, w2_ref, b2_ref, out_ref, l1pad):
    def xs1(sy, sx):
        return z1_ref[0, 1 + sy:17 + sy, 1 + sx:17 + sx, :].reshape(256, _C1)

    y1 = _deconv_planes(xs1, w1_ref, b1_ref, True, 16)
    l1pad[...] = jnp.zeros_like(l1pad)
    l1pad[1:33, 1:33, :] = y1.astype(jnp.bfloat16)

    cache = {}

    def xs2(sy, sx):
        if (sy, sx) not in cache:
            cache[(sy, sx)] = (
                l1pad[1 + sy:33 + sy, 1 + sx:33 + sx, :].reshape(1024, _C1))
        return cache[(sy, sx)]

    pooled = None
    for p in range(4):
        acc = None
        for t, (tp, sy, sx, _, _) in enumerate(_DTAPS):
            if tp != p:
                continue
            d = jnp.dot(xs2(sy, sx), w2_ref[t],
                        preferred_element_type=jnp.float32)
            acc = d if acc is None else acc + d
        pm = jnp.max(acc, axis=0, keepdims=True)
        pooled = pm if pooled is None else jnp.maximum(pooled, pm)
    out_ref[0] = pooled + b2_ref[...]


def _tail_kernel(pool_ref, wg_ref, bg_ref, out_ref):
    x = pool_ref[...].reshape(-1, _CK)
    x = jnp.where(x >= 0, x, 0.01 * x)
    y = jnp.dot(x, wg_ref[...], preferred_element_type=jnp.float32)
    y = y + bg_ref[...]
    a = y[:, 0 * _GM:1 * _GM]
    b = y[:, 1 * _GM:2 * _GM]
    c = y[:, 2 * _GM:3 * _GM]
    m = jnp.maximum(a, jnp.maximum(b, c))
    ea = jnp.exp(a - m)
    eb = jnp.exp(b - m)
    ec = jnp.exp(c - m)
    s = ea + eb + ec
    out_ref[...] = jnp.concatenate([ea / s, eb / s, ec / s], axis=1)


_VMEM_LIMIT = 60000 * 1024


def _full(shape):
    return pl.BlockSpec(shape, lambda b: (0,) * len(shape))


def kernel(z1, gs_w1, gs_b1, gs_w2, gs_b2, gs_w3, gs_b3,
           gm_w1, gm_b1, gm_w2, gm_b2, gm_w3, gm_b3,
           gw_w1, gw_b1, gw_w2, gw_b2, gw_w3, gw_b3):
    nb = z1.shape[0]
    bf = jnp.bfloat16
    z1p = jnp.pad(z1.transpose(0, 2, 3, 1),
                  ((0, 0), (1, 1), (1, 1), (0, 0))).astype(bf)

    def branch(w1, b1, w2, b2, w3, b3, leaky, final_relu):
        w1t = _build_deconv_taps(w1).astype(bf)
        w2t = _build_deconv_taps(w2).astype(bf)
        w3t = _build_conv_taps(w3).astype(bf)
        out = pl.pallas_call(
            functools.partial(_branch_kernel, leaky=leaky,
                              final_relu=final_relu),
            grid=(nb,),
            in_specs=[
                pl.BlockSpec((1, 18, 18, _C1), lambda b: (b, 0, 0, 0)),
                _full((25, _C1, _C1)), _full((1, _C1)),
                _full((25, _C1, _C1)), _full((1, _C1)),
                _full((25, _CK, _C1)), _full((_CK, 1)),
            ],
            out_specs=pl.BlockSpec((1, _CK, 4096), lambda b: (b, 0, 0)),
            out_shape=jax.ShapeDtypeStruct((nb, _CK, 4096), jnp.float32),
            scratch_shapes=[
                pltpu.VMEM((34, 34, _C1), bf),
                pltpu.VMEM((68, 68, _C1), bf),
            ],
            compiler_params=pltpu.CompilerParams(
                dimension_semantics=("arbitrary",),
                vmem_limit_bytes=_VMEM_LIMIT,
            ),
            name="gmm_branch",
        )(z1p, w1t, b1.reshape(1, _C1), w2t, b2.reshape(1, _C1),
          w3t, b3.reshape(_CK, 1))
        return out.reshape(nb, _CK, 64, 64)

    sigma = branch(gs_w1, gs_b1, gs_w2, gs_b2, gs_w3, gs_b3, False, True)
    means = branch(gm_w1, gm_b1, gm_w2, gm_b2, gm_w3, gm_b3, True, False)

    w1t = _build_deconv_taps(gw_w1).astype(bf)
    w2t = _build_deconv_taps(gw_w2).astype(bf)
    pooled = pl.pallas_call(
        _gw_kernel,
        grid=(nb,),
        in_specs=[
            pl.BlockSpec((1, 18, 18, _C1), lambda b: (b, 0, 0, 0)),
            _full((25, _C1, _C1)), _full((1, _C1)),
            _full((25, _C1, _CK)), _full((1, _CK)),
        ],
        out_specs=pl.BlockSpec((1, 1, _CK), lambda b: (b, 0, 0)),
        out_shape=jax.ShapeDtypeStruct((nb, 1, _CK), jnp.float32),
        scratch_shapes=[pltpu.VMEM((34, 34, _C1), bf)],
        compiler_params=pltpu.CompilerParams(
            dimension_semantics=("arbitrary",),
            vmem_limit_bytes=_VMEM_LIMIT,
        ),
        name="gmm_gw_pool",
    )(z1p, w1t, gw_b1.reshape(1, _C1), w2t, gw_b2.reshape(1, _CK))

    weights = pl.pallas_call(
        _tail_kernel,
        in_specs=[
            pl.BlockSpec((nb, 1, _CK), lambda: (0, 0, 0)),
            pl.BlockSpec((_CK, _CK), lambda: (0, 0)),
            pl.BlockSpec((1, _CK), lambda: (0, 0)),
        ],
        out_specs=pl.BlockSpec((nb, _CK), lambda: (0, 0)),
        out_shape=jax.ShapeDtypeStruct((nb, _CK), jnp.float32),
        compiler_params=pltpu.CompilerParams(
            vmem_limit_bytes=_VMEM_LIMIT,
        ),
        name="gmm_gw_tail",
    )(pooled, gw_w3[:, :, 0, 0].transpose(1, 0), gw_b3.reshape(1, _CK))

    return sigma, means, weights.reshape(nb, _CK, 1, 1)


# speedup vs baseline: 1.3437x; 1.3437x over previous
import functools

import numpy as np
import jax
import jax.numpy as jnp
from jax import lax
from jax.experimental import pallas as pl
from jax.experimental.pallas import tpu as pltpu
from jax.sharding import Mesh, NamedSharding, PartitionSpec as P

_C1 = 192
_CK = 576
_GM = 192

_PMAP = ({-1: 0, 0: 2, 1: 4}, {0: 1, 1: 3})

_DTAPS = [
    (2 * a + b, sy, sx, _PMAP[a][sy], _PMAP[b][sx])
    for a in (0, 1) for b in (0, 1)
    for sy in (-1, 0, 1) if sy in _PMAP[a]
    for sx in (-1, 0, 1) if sx in _PMAP[b]
]

_CTAPS = [(dy, dx) for dy in range(5) for dx in range(5)]


def _build_deconv_taps(w):
    return jnp.stack([w[:, :, 4 - p, 4 - q] for (_, _, _, p, q) in _DTAPS])


def _build_conv_taps(w):
    return jnp.stack([w[:, :, dy, dx] for (dy, dx) in _CTAPS])


def _act(x, leaky):
    return jnp.where(x >= 0, x, 0.01 * x) if leaky else jnp.maximum(x, 0.0)


def _deconv_planes(xs, w_ref, b_ref, leaky, h):
    cache = {}

    def get(sy, sx):
        if (sy, sx) not in cache:
            cache[(sy, sx)] = xs(sy, sx)
        return cache[(sy, sx)]

    planes = []
    for p in range(4):
        acc = None
        for t, (tp, sy, sx, _, _) in enumerate(_DTAPS):
            if tp != p:
                continue
            d = jnp.dot(get(sy, sx), w_ref[t],
                        preferred_element_type=jnp.float32)
            acc = d if acc is None else acc + d
        acc = _act(acc + b_ref[...], leaky)
        planes.append(acc.reshape(h, h, -1))
    row_e = jnp.stack([planes[0], planes[1]], axis=2).reshape(h, 2 * h, -1)
    row_o = jnp.stack([planes[2], planes[3]], axis=2).reshape(h, 2 * h, -1)
    return jnp.stack([row_e, row_o], axis=1).reshape(2 * h, 2 * h, -1)


def _branch_kernel(z1_ref, w1_ref, b1_ref, w2_ref, b2_ref, w3_ref, b3_ref,
                   out_ref, l1pad, l2pad, *, leaky, final_relu):
    def xs1(sy, sx):
        return z1_ref[0, 1 + sy:17 + sy, 1 + sx:17 + sx, :].reshape(256, _C1)

    y1 = _deconv_planes(xs1, w1_ref, b1_ref, leaky, 16)
    l1pad[...] = jnp.zeros_like(l1pad)
    l1pad[1:33, 1:33, :] = y1.astype(jnp.bfloat16)

    def xs2(sy, sx):
        return l1pad[1 + sy:33 + sy, 1 + sx:33 + sx, :].reshape(1024, _C1)

    y2 = _deconv_planes(xs2, w2_ref, b2_ref, leaky, 32)
    l2pad[...] = jnp.zeros_like(l2pad)
    l2pad[2:66, 2:66, :] = y2.astype(jnp.bfloat16)

    acc = None
    for t, (dy, dx) in enumerate(_CTAPS):
        xv = l2pad[dy:dy + 64, dx:dx + 64, :].reshape(4096, _C1)
        d = lax.dot_general(w3_ref[t], xv, (((1,), (1,)), ((), ())),
                            preferred_element_type=jnp.float32)
        acc = d if acc is None else acc + d
    acc = acc + b3_ref[...]
    if final_relu:
        acc = jnp.maximum(acc, 0.0)
    out_ref[0] = acc


def _gw_kernel(z1_ref, w1_ref, b1_ref, w2_ref, b2_ref, out_ref, l1pad):
    def xs1(sy, sx):
        return z1_ref[0, 1 + sy:17 + sy, 1 + sx:17 + sx, :].reshape(256, _C1)

    y1 = _deconv_planes(xs1, w1_ref, b1_ref, True, 16)
    l1pad[...] = jnp.zeros_like(l1pad)
    l1pad[1:33, 1:33, :] = y1.astype(jnp.bfloat16)

    cache = {}

    def xs2(sy, sx):
        if (sy, sx) not in cache:
            cache[(sy, sx)] = (
                l1pad[1 + sy:33 + sy, 1 + sx:33 + sx, :].reshape(1024, _C1))
        return cache[(sy, sx)]

    pooled = None
    for p in range(4):
        acc = None
        for t, (tp, sy, sx, _, _) in enumerate(_DTAPS):
            if tp != p:
                continue
            d = jnp.dot(xs2(sy, sx), w2_ref[t],
                        preferred_element_type=jnp.float32)
            acc = d if acc is None else acc + d
        pm = jnp.max(acc, axis=0, keepdims=True)
        pooled = pm if pooled is None else jnp.maximum(pooled, pm)
    out_ref[0] = pooled + b2_ref[...]


def _tail_kernel(pool_ref, wg_ref, bg_ref, out_ref):
    x = pool_ref[...].reshape(-1, _CK)
    x = jnp.where(x >= 0, x, 0.01 * x)
    y = jnp.dot(x, wg_ref[...], preferred_element_type=jnp.float32)
    y = y + bg_ref[...]
    a = y[:, 0 * _GM:1 * _GM]
    b = y[:, 1 * _GM:2 * _GM]
    c = y[:, 2 * _GM:3 * _GM]
    m = jnp.maximum(a, jnp.maximum(b, c))
    ea = jnp.exp(a - m)
    eb = jnp.exp(b - m)
    ec = jnp.exp(c - m)
    s = ea + eb + ec
    out_ref[...] = jnp.concatenate([ea / s, eb / s, ec / s], axis=1)


_VMEM_LIMIT = 60000 * 1024


def _full(shape):
    return pl.BlockSpec(shape, lambda b: (0,) * len(shape))


def _pipeline(z1, gs_w1, gs_b1, gs_w2, gs_b2, gs_w3, gs_b3,
              gm_w1, gm_b1, gm_w2, gm_b2, gm_w3, gm_b3,
              gw_w1, gw_b1, gw_w2, gw_b2, gw_w3, gw_b3):
    nb = z1.shape[0]
    bf = jnp.bfloat16
    z1p = jnp.pad(z1.transpose(0, 2, 3, 1),
                  ((0, 0), (1, 1), (1, 1), (0, 0))).astype(bf)

    def branch(w1, b1, w2, b2, w3, b3, leaky, final_relu):
        w1t = _build_deconv_taps(w1).astype(bf)
        w2t = _build_deconv_taps(w2).astype(bf)
        w3t = _build_conv_taps(w3).astype(bf)
        out = pl.pallas_call(
            functools.partial(_branch_kernel, leaky=leaky,
                              final_relu=final_relu),
            grid=(nb,),
            in_specs=[
                pl.BlockSpec((1, 18, 18, _C1), lambda b: (b, 0, 0, 0)),
                _full((25, _C1, _C1)), _full((1, _C1)),
                _full((25, _C1, _C1)), _full((1, _C1)),
                _full((25, _CK, _C1)), _full((_CK, 1)),
            ],
            out_specs=pl.BlockSpec((1, _CK, 4096), lambda b: (b, 0, 0)),
            out_shape=jax.ShapeDtypeStruct((nb, _CK, 4096), jnp.float32),
            scratch_shapes=[
                pltpu.VMEM((34, 34, _C1), bf),
                pltpu.VMEM((68, 68, _C1), bf),
            ],
            compiler_params=pltpu.CompilerParams(
                dimension_semantics=("arbitrary",),
                vmem_limit_bytes=_VMEM_LIMIT,
            ),
            name="gmm_branch",
        )(z1p, w1t, b1.reshape(1, _C1), w2t, b2.reshape(1, _C1),
          w3t, b3.reshape(_CK, 1))
        return out.reshape(nb, _CK, 64, 64)

    sigma = branch(gs_w1, gs_b1, gs_w2, gs_b2, gs_w3, gs_b3, False, True)
    means = branch(gm_w1, gm_b1, gm_w2, gm_b2, gm_w3, gm_b3, True, False)

    w1t = _build_deconv_taps(gw_w1).astype(bf)
    w2t = _build_deconv_taps(gw_w2).astype(bf)
    pooled = pl.pallas_call(
        _gw_kernel,
        grid=(nb,),
        in_specs=[
            pl.BlockSpec((1, 18, 18, _C1), lambda b: (b, 0, 0, 0)),
            _full((25, _C1, _C1)), _full((1, _C1)),
            _full((25, _C1, _CK)), _full((1, _CK)),
        ],
        out_specs=pl.BlockSpec((1, 1, _CK), lambda b: (b, 0, 0)),
        out_shape=jax.ShapeDtypeStruct((nb, 1, _CK), jnp.float32),
        scratch_shapes=[pltpu.VMEM((34, 34, _C1), bf)],
        compiler_params=pltpu.CompilerParams(
            dimension_semantics=("arbitrary",),
            vmem_limit_bytes=_VMEM_LIMIT,
        ),
        name="gmm_gw_pool",
    )(z1p, w1t, gw_b1.reshape(1, _C1), w2t, gw_b2.reshape(1, _CK))

    weights = pl.pallas_call(
        _tail_kernel,
        in_specs=[
            pl.BlockSpec((nb, 1, _CK), lambda: (0, 0, 0)),
            pl.BlockSpec((_CK, _CK), lambda: (0, 0)),
            pl.BlockSpec((1, _CK), lambda: (0, 0)),
        ],
        out_specs=pl.BlockSpec((nb, _CK), lambda: (0, 0)),
        out_shape=jax.ShapeDtypeStruct((nb, _CK), jnp.float32),
        compiler_params=pltpu.CompilerParams(
            vmem_limit_bytes=_VMEM_LIMIT,
        ),
        name="gmm_gw_tail",
    )(pooled, gw_w3[:, :, 0, 0].transpose(1, 0), gw_b3.reshape(1, _CK))

    return sigma, means, weights.reshape(nb, _CK, 1, 1)


def kernel(z1, gs_w1, gs_b1, gs_w2, gs_b2, gs_w3, gs_b3,
           gm_w1, gm_b1, gm_w2, gm_b2, gm_w3, gm_b3,
           gw_w1, gw_b1, gw_w2, gw_b2, gw_w3, gw_b3):
    args = (z1, gs_w1, gs_b1, gs_w2, gs_b2, gs_w3, gs_b3,
            gm_w1, gm_b1, gm_w2, gm_b2, gm_w3, gm_b3,
            gw_w1, gw_b1, gw_w2, gw_b2, gw_w3, gw_b3)
    devs = jax.devices()
    nd = 2 if len(devs) >= 2 and z1.shape[0] % 2 == 0 else 1
    if nd == 1:
        return _pipeline(*args)
    mesh = Mesh(np.array(devs[:nd]), ("d",))
    in_specs = (P("d"),) + (P(),) * 18
    outs = jax.shard_map(_pipeline, mesh=mesh, in_specs=in_specs,
                         out_specs=P("d"), check_vma=False)(*args)
    return jax.lax.with_sharding_constraint(
        outs, NamedSharding(mesh, P()))


# speedup vs baseline: 1.7763x; 1.3220x over previous
import functools

import numpy as np
import jax
import jax.numpy as jnp
from jax import lax
from jax.experimental import pallas as pl
from jax.experimental.pallas import tpu as pltpu
from jax.sharding import Mesh, PartitionSpec as P

_C1 = 192
_CK = 576
_GM = 192

_PMAP = ({-1: 0, 0: 2, 1: 4}, {0: 1, 1: 3})

_DTAPS = [
    (2 * a + b, sy, sx, _PMAP[a][sy], _PMAP[b][sx])
    for a in (0, 1) for b in (0, 1)
    for sy in (-1, 0, 1) if sy in _PMAP[a]
    for sx in (-1, 0, 1) if sx in _PMAP[b]
]

_CTAPS = [(dy, dx) for dy in range(5) for dx in range(5)]


def _build_deconv_taps(w):
    return jnp.stack([w[:, :, 4 - p, 4 - q] for (_, _, _, p, q) in _DTAPS])


def _build_conv_taps(w):
    return jnp.stack([w[:, :, dy, dx] for (dy, dx) in _CTAPS])


def _act(x, leaky):
    return jnp.where(x >= 0, x, 0.01 * x) if leaky else jnp.maximum(x, 0.0)


def _deconv_planes(xs, w_ref, b_ref, leaky, h):
    cache = {}

    def get(sy, sx):
        if (sy, sx) not in cache:
            cache[(sy, sx)] = xs(sy, sx)
        return cache[(sy, sx)]

    planes = []
    for p in range(4):
        acc = None
        for t, (tp, sy, sx, _, _) in enumerate(_DTAPS):
            if tp != p:
                continue
            d = jnp.dot(get(sy, sx), w_ref[t],
                        preferred_element_type=jnp.float32)
            acc = d if acc is None else acc + d
        acc = _act(acc + b_ref[...], leaky)
        planes.append(acc.reshape(h, h, -1))
    row_e = jnp.stack([planes[0], planes[1]], axis=2).reshape(h, 2 * h, -1)
    row_o = jnp.stack([planes[2], planes[3]], axis=2).reshape(h, 2 * h, -1)
    return jnp.stack([row_e, row_o], axis=1).reshape(2 * h, 2 * h, -1)


def _branch_kernel(z1_ref, w1_ref, b1_ref, w2_ref, b2_ref, w3_ref, b3_ref,
                   out_ref, l1pad, l2pad, *, leaky, final_relu):
    def xs1(sy, sx):
        return z1_ref[0, 1 + sy:17 + sy, 1 + sx:17 + sx, :].reshape(256, _C1)

    y1 = _deconv_planes(xs1, w1_ref, b1_ref, leaky, 16)
    l1pad[...] = jnp.zeros_like(l1pad)
    l1pad[1:33, 1:33, :] = y1.astype(jnp.bfloat16)

    def xs2(sy, sx):
        return l1pad[1 + sy:33 + sy, 1 + sx:33 + sx, :].reshape(1024, _C1)

    y2 = _deconv_planes(xs2, w2_ref, b2_ref, leaky, 32)
    l2pad[...] = jnp.zeros_like(l2pad)
    l2pad[2:66, 2:66, :] = y2.astype(jnp.bfloat16)

    acc = None
    for t, (dy, dx) in enumerate(_CTAPS):
        xv = l2pad[dy:dy + 64, dx:dx + 64, :].reshape(4096, _C1)
        d = lax.dot_general(w3_ref[t], xv, (((1,), (1,)), ((), ())),
                            preferred_element_type=jnp.float32)
        acc = d if acc is None else acc + d
    acc = acc + b3_ref[...]
    if final_relu:
        acc = jnp.maximum(acc, 0.0)
    out_ref[0] = acc


def _gw_kernel(z1_ref, w1_ref, b1_ref, w2_ref, b2_ref, out_ref, l1pad):
    def xs1(sy, sx):
        return z1_ref[0, 1 + sy:17 + sy, 1 + sx:17 + sx, :].reshape(256, _C1)

    y1 = _deconv_planes(xs1, w1_ref, b1_ref, True, 16)
    l1pad[...] = jnp.zeros_like(l1pad)
    l1pad[1:33, 1:33, :] = y1.astype(jnp.bfloat16)

    cache = {}

    def xs2(sy, sx):
        if (sy, sx) not in cache:
            cache[(sy, sx)] = (
                l1pad[1 + sy:33 + sy, 1 + sx:33 + sx, :].reshape(1024, _C1))
        return cache[(sy, sx)]

    pooled = None
    for p in range(4):
        acc = None
        for t, (tp, sy, sx, _, _) in enumerate(_DTAPS):
            if tp != p:
                continue
            d = jnp.dot(xs2(sy, sx), w2_ref[t],
                        preferred_element_type=jnp.float32)
            acc = d if acc is None else acc + d
        pm = jnp.max(acc, axis=0, keepdims=True)
        pooled = pm if pooled is None else jnp.maximum(pooled, pm)
    out_ref[0] = pooled + b2_ref[...]


def _tail_kernel(pool_ref, wg_ref, bg_ref, out_ref):
    x = pool_ref[...].reshape(-1, _CK)
    x = jnp.where(x >= 0, x, 0.01 * x)
    y = jnp.dot(x, wg_ref[...], preferred_element_type=jnp.float32)
    y = y + bg_ref[...]
    a = y[:, 0 * _GM:1 * _GM]
    b = y[:, 1 * _GM:2 * _GM]
    c = y[:, 2 * _GM:3 * _GM]
    m = jnp.maximum(a, jnp.maximum(b, c))
    ea = jnp.exp(a - m)
    eb = jnp.exp(b - m)
    ec = jnp.exp(c - m)
    s = ea + eb + ec
    out_ref[...] = jnp.concatenate([ea / s, eb / s, ec / s], axis=1)


_VMEM_LIMIT = 60000 * 1024


def _full(shape):
    return pl.BlockSpec(shape, lambda b: (0,) * len(shape))


def _pipeline(z1, gs_w1, gs_b1, gs_w2, gs_b2, gs_w3, gs_b3,
              gm_w1, gm_b1, gm_w2, gm_b2, gm_w3, gm_b3,
              gw_w1, gw_b1, gw_w2, gw_b2, gw_w3, gw_b3):
    nb = z1.shape[0]
    bf = jnp.bfloat16
    z1p = jnp.pad(z1.transpose(0, 2, 3, 1),
                  ((0, 0), (1, 1), (1, 1), (0, 0))).astype(bf)

    def branch(w1, b1, w2, b2, w3, b3, leaky, final_relu):
        w1t = _build_deconv_taps(w1).astype(bf)
        w2t = _build_deconv_taps(w2).astype(bf)
        w3t = _build_conv_taps(w3).astype(bf)
        out = pl.pallas_call(
            functools.partial(_branch_kernel, leaky=leaky,
                              final_relu=final_relu),
            grid=(nb,),
            in_specs=[
                pl.BlockSpec((1, 18, 18, _C1), lambda b: (b, 0, 0, 0)),
                _full((25, _C1, _C1)), _full((1, _C1)),
                _full((25, _C1, _C1)), _full((1, _C1)),
                _full((25, _CK, _C1)), _full((_CK, 1)),
            ],
            out_specs=pl.BlockSpec((1, _CK, 4096), lambda b: (b, 0, 0)),
            out_shape=jax.ShapeDtypeStruct((nb, _CK, 4096), jnp.float32),
            scratch_shapes=[
                pltpu.VMEM((34, 34, _C1), bf),
                pltpu.VMEM((68, 68, _C1), bf),
            ],
            compiler_params=pltpu.CompilerParams(
                dimension_semantics=("arbitrary",),
                vmem_limit_bytes=_VMEM_LIMIT,
            ),
            name="gmm_branch",
        )(z1p, w1t, b1.reshape(1, _C1), w2t, b2.reshape(1, _C1),
          w3t, b3.reshape(_CK, 1))
        return out.reshape(nb, _CK, 64, 64)

    sigma = branch(gs_w1, gs_b1, gs_w2, gs_b2, gs_w3, gs_b3, False, True)
    means = branch(gm_w1, gm_b1, gm_w2, gm_b2, gm_w3, gm_b3, True, False)

    w1t = _build_deconv_taps(gw_w1).astype(bf)
    w2t = _build_deconv_taps(gw_w2).astype(bf)
    pooled = pl.pallas_call(
        _gw_kernel,
        grid=(nb,),
        in_specs=[
            pl.BlockSpec((1, 18, 18, _C1), lambda b: (b, 0, 0, 0)),
            _full((25, _C1, _C1)), _full((1, _C1)),
            _full((25, _C1, _CK)), _full((1, _CK)),
        ],
        out_specs=pl.BlockSpec((1, 1, _CK), lambda b: (b, 0, 0)),
        out_shape=jax.ShapeDtypeStruct((nb, 1, _CK), jnp.float32),
        scratch_shapes=[pltpu.VMEM((34, 34, _C1), bf)],
        compiler_params=pltpu.CompilerParams(
            dimension_semantics=("arbitrary",),
            vmem_limit_bytes=_VMEM_LIMIT,
        ),
        name="gmm_gw_pool",
    )(z1p, w1t, gw_b1.reshape(1, _C1), w2t, gw_b2.reshape(1, _CK))

    weights = pl.pallas_call(
        _tail_kernel,
        in_specs=[
            pl.BlockSpec((nb, 1, _CK), lambda: (0, 0, 0)),
            pl.BlockSpec((_CK, _CK), lambda: (0, 0)),
            pl.BlockSpec((1, _CK), lambda: (0, 0)),
        ],
        out_specs=pl.BlockSpec((nb, _CK), lambda: (0, 0)),
        out_shape=jax.ShapeDtypeStruct((nb, _CK), jnp.float32),
        compiler_params=pltpu.CompilerParams(
            vmem_limit_bytes=_VMEM_LIMIT,
        ),
        name="gmm_gw_tail",
    )(pooled, gw_w3[:, :, 0, 0].transpose(1, 0), gw_b3.reshape(1, _CK))

    return sigma, means, weights.reshape(nb, _CK, 1, 1)


def kernel(z1, gs_w1, gs_b1, gs_w2, gs_b2, gs_w3, gs_b3,
           gm_w1, gm_b1, gm_w2, gm_b2, gm_w3, gm_b3,
           gw_w1, gw_b1, gw_w2, gw_b2, gw_w3, gw_b3):
    args = (z1, gs_w1, gs_b1, gs_w2, gs_b2, gs_w3, gs_b3,
            gm_w1, gm_b1, gm_w2, gm_b2, gm_w3, gm_b3,
            gw_w1, gw_b1, gw_w2, gw_b2, gw_w3, gw_b3)
    devs = jax.devices()
    nd = 2 if len(devs) >= 2 and z1.shape[0] % 2 == 0 else 1
    if nd == 1:
        return _pipeline(*args)
    mesh = Mesh(np.array(devs[:nd]), ("d",))
    in_specs = (P("d"),) + (P(),) * 18
    return jax.shard_map(_pipeline, mesh=mesh, in_specs=in_specs,
                         out_specs=P("d"), check_vma=False)(*args)


# speedup vs baseline: 2.4928x; 1.4034x over previous
import functools

import numpy as np
import jax
import jax.numpy as jnp
from jax import lax
from jax.experimental import pallas as pl
from jax.experimental.pallas import tpu as pltpu
from jax.sharding import Mesh, PartitionSpec as P

_C1 = 192
_CK = 576
_GM = 192

_PMAP = ({-1: 0, 0: 2, 1: 4}, {0: 1, 1: 3})

_TAPS_A = [(sy, sx) for sy in (-1, 0, 1) for sx in (-1, 0, 1)]
_TAPS_B = [(sy, sx) for sy in (0, 1) for sx in (-1, 0, 1)]

_DTAPS = [
    (2 * a + b, sy, sx)
    for a in (0, 1) for b in (0, 1)
    for sy in (-1, 0, 1) if sy in _PMAP[a]
    for sx in (-1, 0, 1) if sx in _PMAP[b]
]


def _dtap_mat(w, a, b, sy, sx):
    cin, cout = w.shape[0], w.shape[1]
    p = _PMAP[a].get(sy)
    q = _PMAP[b].get(sx)
    if p is None or q is None:
        return jnp.zeros((cin, cout), w.dtype)
    return w[:, :, 4 - p, 4 - q]


def _build_pair_taps(w):
    wa = jnp.stack([
        jnp.concatenate([_dtap_mat(w, 0, 0, sy, sx),
                         _dtap_mat(w, 0, 1, sy, sx)], axis=1)
        for (sy, sx) in _TAPS_A])
    wb = jnp.stack([
        jnp.concatenate([_dtap_mat(w, 1, 0, sy, sx),
                         _dtap_mat(w, 1, 1, sy, sx)], axis=1)
        for (sy, sx) in _TAPS_B])
    return wa, wb


def _build_gw_l2_taps(w):
    return jnp.stack([_dtap_mat(w, pl_ // 2, pl_ % 2, sy, sx)
                      for (pl_, sy, sx) in _DTAPS])


def _build_conv_rowtaps(w):
    return jnp.stack([
        jnp.concatenate([w[:, :, dy, dx] for dx in range(5)], axis=1)
        for dy in range(5)])


def _act(x, leaky):
    return jnp.where(x >= 0, x, 0.01 * x) if leaky else jnp.maximum(x, 0.0)


def _deconv_pairs(xs, wa_ref, wb_ref, b2_ref, leaky, h):
    cache = {}

    def get(sy, sx):
        if (sy, sx) not in cache:
            cache[(sy, sx)] = xs(sy, sx)
        return cache[(sy, sx)]

    acc_a = None
    for t, (sy, sx) in enumerate(_TAPS_A):
        d = jnp.dot(get(sy, sx), wa_ref[t], preferred_element_type=jnp.float32)
        acc_a = d if acc_a is None else acc_a + d
    acc_b = None
    for t, (sy, sx) in enumerate(_TAPS_B):
        d = jnp.dot(get(sy, sx), wb_ref[t], preferred_element_type=jnp.float32)
        acc_b = d if acc_b is None else acc_b + d
    acc_a = _act(acc_a + b2_ref[...], leaky)
    acc_b = _act(acc_b + b2_ref[...], leaky)
    c = acc_a.shape[1] // 2
    ee = acc_a[:, :c].reshape(h, h, c)
    eo = acc_a[:, c:].reshape(h, h, c)
    oe = acc_b[:, :c].reshape(h, h, c)
    oo = acc_b[:, c:].reshape(h, h, c)
    row_e = jnp.stack([ee, eo], axis=2).reshape(h, 2 * h, c)
    row_o = jnp.stack([oe, oo], axis=2).reshape(h, 2 * h, c)
    return jnp.stack([row_e, row_o], axis=1).reshape(2 * h, 2 * h, c)


def _branch_kernel(z1_ref, w1a_ref, w1b_ref, b1_ref, w2a_ref, w2b_ref, b2_ref,
                   w3_ref, b3_ref, out_ref, l1pad, zdx, *, leaky, final_relu):
    j = pl.program_id(1)

    @pl.when((pl.program_id(0) == 0) & (j == 0))
    def _init():
        l1pad[...] = jnp.zeros_like(l1pad)
        zdx[...] = jnp.zeros_like(zdx)

    @pl.when(j == 0)
    def _front():
        def xs1(sy, sx):
            return z1_ref[0, 1 + sy:17 + sy, 1 + sx:17 + sx, :].reshape(
                256, _C1)

        y1 = _deconv_pairs(xs1, w1a_ref, w1b_ref, b1_ref, leaky, 16)
        l1pad[1:33, 1:33, :] = y1.astype(jnp.bfloat16)

        def xs2(sy, sx):
            return l1pad[1 + sy:33 + sy, 1 + sx:33 + sx, :].reshape(1024, _C1)

        y2 = _deconv_pairs(xs2, w2a_ref, w2b_ref, b2_ref, leaky, 32)
        y2 = y2.astype(jnp.bfloat16)
        for g in range(5):
            cs = max(0, 2 - g)
            ce = min(64, 66 - g)
            zdx[2:66, cs:ce, g * _C1:(g + 1) * _C1] = (
                y2[:, cs + g - 2:ce + g - 2, :])

    acc = None
    for dy in range(5):
        xv = zdx[pl.ds(j * 32 + dy, 32), :, :].reshape(2048, 5 * _C1)
        d = lax.dot_general(w3_ref[dy], xv, (((1,), (1,)), ((), ())),
                            preferred_element_type=jnp.float32)
        acc = d if acc is None else acc + d
    acc = acc + b3_ref[...]
    if final_relu:
        acc = jnp.maximum(acc, 0.0)
    out_ref[0] = acc


def _gw_kernel(z1_ref, w1a_ref, w1b_ref, b1_ref, w2_ref, b2_ref, out_ref,
               l1pad):
    @pl.when(pl.program_id(0) == 0)
    def _init():
        l1pad[...] = jnp.zeros_like(l1pad)

    def xs1(sy, sx):
        return z1_ref[0, 1 + sy:17 + sy, 1 + sx:17 + sx, :].reshape(256, _C1)

    y1 = _deconv_pairs(xs1, w1a_ref, w1b_ref, b1_ref, True, 16)
    l1pad[1:33, 1:33, :] = y1.astype(jnp.bfloat16)

    cache = {}

    def xs2(sy, sx):
        if (sy, sx) not in cache:
            cache[(sy, sx)] = (
                l1pad[1 + sy:33 + sy, 1 + sx:33 + sx, :].reshape(1024, _C1))
        return cache[(sy, sx)]

    pooled = None
    for p in range(4):
        acc = None
        for t, (tp, sy, sx) in enumerate(_DTAPS):
            if tp != p:
                continue
            d = jnp.dot(xs2(sy, sx), w2_ref[t],
                        preferred_element_type=jnp.float32)
            acc = d if acc is None else acc + d
        pm = jnp.max(acc, axis=0, keepdims=True)
        pooled = pm if pooled is None else jnp.maximum(pooled, pm)
    out_ref[0] = pooled + b2_ref[...]


def _tail_kernel(pool_ref, wg_ref, bg_ref, out_ref):
    x = pool_ref[...].reshape(-1, _CK)
    x = jnp.where(x >= 0, x, 0.01 * x)
    y = jnp.dot(x, wg_ref[...], preferred_element_type=jnp.float32)
    y = y + bg_ref[...]
    a = y[:, 0 * _GM:1 * _GM]
    b = y[:, 1 * _GM:2 * _GM]
    c = y[:, 2 * _GM:3 * _GM]
    m = jnp.maximum(a, jnp.maximum(b, c))
    ea = jnp.exp(a - m)
    eb = jnp.exp(b - m)
    ec = jnp.exp(c - m)
    s = ea + eb + ec
    out_ref[...] = jnp.concatenate([ea / s, eb / s, ec / s], axis=1)


_VMEM_LIMIT = 60000 * 1024


def _full(shape):
    n = len(shape)
    return pl.BlockSpec(shape, lambda *_: (0,) * n)


def _pipeline(z1, gs_w1, gs_b1, gs_w2, gs_b2, gs_w3, gs_b3,
              gm_w1, gm_b1, gm_w2, gm_b2, gm_w3, gm_b3,
              gw_w1, gw_b1, gw_w2, gw_b2, gw_w3, gw_b3):
    nb = z1.shape[0]
    bf = jnp.bfloat16
    z1p = jnp.pad(z1.transpose(0, 2, 3, 1),
                  ((0, 0), (1, 1), (1, 1), (0, 0))).astype(bf)

    def dbl(bias):
        return jnp.concatenate([bias, bias]).reshape(1, 2 * bias.shape[0])

    def branch(w1, b1, w2, b2, w3, b3, leaky, final_relu):
        w1a, w1b = _build_pair_taps(w1)
        w2a, w2b = _build_pair_taps(w2)
        w3r = _build_conv_rowtaps(w3).astype(bf)
        out = pl.pallas_call(
            functools.partial(_branch_kernel, leaky=leaky,
                              final_relu=final_relu),
            grid=(nb, 2),
            in_specs=[
                pl.BlockSpec((1, 18, 18, _C1), lambda b, j: (b, 0, 0, 0)),
                _full((9, _C1, 2 * _C1)), _full((6, _C1, 2 * _C1)),
                _full((1, 2 * _C1)),
                _full((9, _C1, 2 * _C1)), _full((6, _C1, 2 * _C1)),
                _full((1, 2 * _C1)),
                _full((5, _CK, 5 * _C1)), _full((_CK, 1)),
            ],
            out_specs=pl.BlockSpec((1, _CK, 2048), lambda b, j: (b, 0, j)),
            out_shape=jax.ShapeDtypeStruct((nb, _CK, 4096), jnp.float32),
            scratch_shapes=[
                pltpu.VMEM((34, 34, _C1), bf),
                pltpu.VMEM((68, 64, 5 * _C1), bf),
            ],
            compiler_params=pltpu.CompilerParams(
                dimension_semantics=("arbitrary", "arbitrary"),
                vmem_limit_bytes=_VMEM_LIMIT,
            ),
            name="gmm_branch",
        )(z1p, w1a.astype(bf), w1b.astype(bf), dbl(b1),
          w2a.astype(bf), w2b.astype(bf), dbl(b2),
          w3r, b3.reshape(_CK, 1))
        return out.reshape(nb, _CK, 64, 64)

    sigma = branch(gs_w1, gs_b1, gs_w2, gs_b2, gs_w3, gs_b3, False, True)
    means = branch(gm_w1, gm_b1, gm_w2, gm_b2, gm_w3, gm_b3, True, False)

    w1a, w1b = _build_pair_taps(gw_w1)
    w2t = _build_gw_l2_taps(gw_w2).astype(bf)
    pooled = pl.pallas_call(
        _gw_kernel,
        grid=(nb,),
        in_specs=[
            pl.BlockSpec((1, 18, 18, _C1), lambda b: (b, 0, 0, 0)),
            _full((9, _C1, 2 * _C1)), _full((6, _C1, 2 * _C1)),
            _full((1, 2 * _C1)),
            _full((25, _C1, _CK)), _full((1, _CK)),
        ],
        out_specs=pl.BlockSpec((1, 1, _CK), lambda b: (b, 0, 0)),
        out_shape=jax.ShapeDtypeStruct((nb, 1, _CK), jnp.float32),
        scratch_shapes=[pltpu.VMEM((34, 34, _C1), bf)],
        compiler_params=pltpu.CompilerParams(
            dimension_semantics=("arbitrary",),
            vmem_limit_bytes=_VMEM_LIMIT,
        ),
        name="gmm_gw_pool",
    )(z1p, w1a.astype(bf), w1b.astype(bf), dbl(gw_b1), w2t,
      gw_b2.reshape(1, _CK))

    weights = pl.pallas_call(
        _tail_kernel,
        in_specs=[
            pl.BlockSpec((nb, 1, _CK), lambda: (0, 0, 0)),
            pl.BlockSpec((_CK, _CK), lambda: (0, 0)),
            pl.BlockSpec((1, _CK), lambda: (0, 0)),
        ],
        out_specs=pl.BlockSpec((nb, _CK), lambda: (0, 0)),
        out_shape=jax.ShapeDtypeStruct((nb, _CK), jnp.float32),
        compiler_params=pltpu.CompilerParams(
            vmem_limit_bytes=_VMEM_LIMIT,
        ),
        name="gmm_gw_tail",
    )(pooled, gw_w3[:, :, 0, 0].transpose(1, 0), gw_b3.reshape(1, _CK))

    return sigma, means, weights.reshape(nb, _CK, 1, 1)


def kernel(z1, gs_w1, gs_b1, gs_w2, gs_b2, gs_w3, gs_b3,
           gm_w1, gm_b1, gm_w2, gm_b2, gm_w3, gm_b3,
           gw_w1, gw_b1, gw_w2, gw_b2, gw_w3, gw_b3):
    args = (z1, gs_w1, gs_b1, gs_w2, gs_b2, gs_w3, gs_b3,
            gm_w1, gm_b1, gm_w2, gm_b2, gm_w3, gm_b3,
            gw_w1, gw_b1, gw_w2, gw_b2, gw_w3, gw_b3)
    devs = jax.devices()
    nd = 2 if len(devs) >= 2 and z1.shape[0] % 2 == 0 else 1
    if nd == 1:
        return _pipeline(*args)
    mesh = Mesh(np.array(devs[:nd]), ("d",))
    in_specs = (P("d"),) + (P(),) * 18
    return jax.shard_map(_pipeline, mesh=mesh, in_specs=in_specs,
                         out_specs=P("d"), check_vma=False)(*args)


# speedup vs baseline: 2.6286x; 1.0545x over previous
import functools

import numpy as np
import jax
import jax.numpy as jnp
from jax import lax
from jax.experimental import pallas as pl
from jax.experimental.pallas import tpu as pltpu
from jax.sharding import Mesh, PartitionSpec as P

_C1 = 192
_CK = 576
_GM = 192

_PMAP = ({-1: 0, 0: 2, 1: 4}, {0: 1, 1: 3})

_TAPS_A = [(sy, sx) for sy in (-1, 0, 1) for sx in (-1, 0, 1)]
_TAPS_B = [(sy, sx) for sy in (0, 1) for sx in (-1, 0, 1)]

def _dtap_mat(w, a, b, sy, sx):
    cin, cout = w.shape[0], w.shape[1]
    p = _PMAP[a].get(sy)
    q = _PMAP[b].get(sx)
    if p is None or q is None:
        return jnp.zeros((cin, cout), w.dtype)
    return w[:, :, 4 - p, 4 - q]


def _build_pair_taps(w):
    wa = jnp.concatenate([
        jnp.concatenate([_dtap_mat(w, 0, 0, sy, sx),
                         _dtap_mat(w, 0, 1, sy, sx)], axis=1)
        for (sy, sx) in _TAPS_A], axis=0)
    wb = jnp.concatenate([
        jnp.concatenate([_dtap_mat(w, 1, 0, sy, sx),
                         _dtap_mat(w, 1, 1, sy, sx)], axis=1)
        for (sy, sx) in _TAPS_B], axis=0)
    return wa, wb


def _build_conv_cat(w):
    return jnp.concatenate(
        [w[:, :, dy, dx] for dy in range(5) for dx in range(5)], axis=1)


def _act(x, leaky):
    return jnp.where(x >= 0, x, 0.01 * x) if leaky else jnp.maximum(x, 0.0)


def _shift_cat(xs):
    x9 = jnp.concatenate([xs(sy, sx) for (sy, sx) in _TAPS_A], axis=1)
    return x9, x9[:, 3 * _C1:]


def _deconv_pairs(xs, wa_ref, wb_ref, b2_ref, leaky, h):
    x9, x6 = _shift_cat(xs)
    acc_a = jnp.dot(x9, wa_ref[...], preferred_element_type=jnp.float32)
    acc_b = jnp.dot(x6, wb_ref[...], preferred_element_type=jnp.float32)
    acc_a = _act(acc_a + b2_ref[...], leaky)
    acc_b = _act(acc_b + b2_ref[...], leaky)
    c = acc_a.shape[1] // 2
    ee = acc_a[:, :c].reshape(h, h, c)
    eo = acc_a[:, c:].reshape(h, h, c)
    oe = acc_b[:, :c].reshape(h, h, c)
    oo = acc_b[:, c:].reshape(h, h, c)
    row_e = jnp.stack([ee, eo], axis=2).reshape(h, 2 * h, c)
    row_o = jnp.stack([oe, oo], axis=2).reshape(h, 2 * h, c)
    return jnp.stack([row_e, row_o], axis=1).reshape(2 * h, 2 * h, c)


def _branch_kernel(z1_ref, w1a_ref, w1b_ref, b1_ref, w2a_ref, w2b_ref, b2_ref,
                   w3_ref, b3_ref, out_ref, l1pad, zdx, *, leaky, final_relu):
    j = pl.program_id(1)

    @pl.when((pl.program_id(0) == 0) & (j == 0))
    def _init():
        l1pad[...] = jnp.zeros_like(l1pad)
        zdx[...] = jnp.zeros_like(zdx)

    @pl.when(j == 0)
    def _front():
        def xs1(sy, sx):
            return z1_ref[0, 1 + sy:17 + sy, 1 + sx:17 + sx, :].reshape(
                256, _C1)

        y1 = _deconv_pairs(xs1, w1a_ref, w1b_ref, b1_ref, leaky, 16)
        l1pad[1:33, 1:33, :] = y1.astype(jnp.bfloat16)

        def xs2(sy, sx):
            return l1pad[1 + sy:33 + sy, 1 + sx:33 + sx, :].reshape(1024, _C1)

        y2 = _deconv_pairs(xs2, w2a_ref, w2b_ref, b2_ref, leaky, 32)
        y2 = y2.astype(jnp.bfloat16)
        for g in range(5):
            cs = max(0, 2 - g)
            ce = min(64, 66 - g)
            zdx[2:66, cs:ce, g * _C1:(g + 1) * _C1] = (
                y2[:, cs + g - 2:ce + g - 2, :])

    xv = jnp.concatenate(
        [zdx[pl.ds(j * 16 + dy, 16), :, :].reshape(1024, 5 * _C1)
         for dy in range(5)], axis=1)
    acc = lax.dot_general(w3_ref[...], xv, (((1,), (1,)), ((), ())),
                          preferred_element_type=jnp.float32)
    acc = acc + b3_ref[...]
    if final_relu:
        acc = jnp.maximum(acc, 0.0)
    out_ref[0] = acc


def _gw_kernel(z1_ref, w1a_ref, w1b_ref, b1_ref, w2a_ref, w2b_ref, b2_ref,
               out_ref, l1pad):
    @pl.when(pl.program_id(0) == 0)
    def _init():
        l1pad[...] = jnp.zeros_like(l1pad)

    def xs1(sy, sx):
        return z1_ref[0, 1 + sy:17 + sy, 1 + sx:17 + sx, :].reshape(256, _C1)

    y1 = _deconv_pairs(xs1, w1a_ref, w1b_ref, b1_ref, True, 16)
    l1pad[1:33, 1:33, :] = y1.astype(jnp.bfloat16)

    def xs2(sy, sx):
        return l1pad[1 + sy:33 + sy, 1 + sx:33 + sx, :].reshape(1024, _C1)

    x9, x6 = _shift_cat(xs2)
    acc_a = jnp.dot(x9, w2a_ref[...], preferred_element_type=jnp.float32)
    acc_b = jnp.dot(x6, w2b_ref[...], preferred_element_type=jnp.float32)
    pm_a = jnp.max(acc_a, axis=0, keepdims=True)
    pm_b = jnp.max(acc_b, axis=0, keepdims=True)
    pooled = jnp.maximum(
        jnp.maximum(pm_a[:, :_CK], pm_a[:, _CK:]),
        jnp.maximum(pm_b[:, :_CK], pm_b[:, _CK:]))
    out_ref[0] = pooled + b2_ref[...]


def _tail_kernel(pool_ref, wg_ref, bg_ref, out_ref):
    x = pool_ref[...].reshape(-1, _CK)
    x = jnp.where(x >= 0, x, 0.01 * x)
    y = jnp.dot(x, wg_ref[...], preferred_element_type=jnp.float32)
    y = y + bg_ref[...]
    a = y[:, 0 * _GM:1 * _GM]
    b = y[:, 1 * _GM:2 * _GM]
    c = y[:, 2 * _GM:3 * _GM]
    m = jnp.maximum(a, jnp.maximum(b, c))
    ea = jnp.exp(a - m)
    eb = jnp.exp(b - m)
    ec = jnp.exp(c - m)
    s = ea + eb + ec
    out_ref[...] = jnp.concatenate([ea / s, eb / s, ec / s], axis=1)


_VMEM_LIMIT = 60000 * 1024


def _full(shape):
    n = len(shape)
    return pl.BlockSpec(shape, lambda *_: (0,) * n)


def _pipeline(z1, gs_w1, gs_b1, gs_w2, gs_b2, gs_w3, gs_b3,
              gm_w1, gm_b1, gm_w2, gm_b2, gm_w3, gm_b3,
              gw_w1, gw_b1, gw_w2, gw_b2, gw_w3, gw_b3):
    nb = z1.shape[0]
    bf = jnp.bfloat16
    z1p = jnp.pad(z1.transpose(0, 2, 3, 1),
                  ((0, 0), (1, 1), (1, 1), (0, 0))).astype(bf)

    def dbl(bias):
        return jnp.concatenate([bias, bias]).reshape(1, 2 * bias.shape[0])

    def branch(w1, b1, w2, b2, w3, b3, leaky, final_relu):
        w1a, w1b = _build_pair_taps(w1)
        w2a, w2b = _build_pair_taps(w2)
        w3r = _build_conv_cat(w3).astype(bf)
        out = pl.pallas_call(
            functools.partial(_branch_kernel, leaky=leaky,
                              final_relu=final_relu),
            grid=(nb, 4),
            in_specs=[
                pl.BlockSpec((1, 18, 18, _C1), lambda b, j: (b, 0, 0, 0)),
                _full((9 * _C1, 2 * _C1)), _full((6 * _C1, 2 * _C1)),
                _full((1, 2 * _C1)),
                _full((9 * _C1, 2 * _C1)), _full((6 * _C1, 2 * _C1)),
                _full((1, 2 * _C1)),
                _full((_CK, 25 * _C1)), _full((_CK, 1)),
            ],
            out_specs=pl.BlockSpec((1, _CK, 1024), lambda b, j: (b, 0, j)),
            out_shape=jax.ShapeDtypeStruct((nb, _CK, 4096), jnp.float32),
            scratch_shapes=[
                pltpu.VMEM((34, 34, _C1), bf),
                pltpu.VMEM((68, 64, 5 * _C1), bf),
            ],
            compiler_params=pltpu.CompilerParams(
                dimension_semantics=("arbitrary", "arbitrary"),
                vmem_limit_bytes=_VMEM_LIMIT,
            ),
            name="gmm_branch",
        )(z1p, w1a.astype(bf), w1b.astype(bf), dbl(b1),
          w2a.astype(bf), w2b.astype(bf), dbl(b2),
          w3r, b3.reshape(_CK, 1))
        return out.reshape(nb, _CK, 64, 64)

    sigma = branch(gs_w1, gs_b1, gs_w2, gs_b2, gs_w3, gs_b3, False, True)
    means = branch(gm_w1, gm_b1, gm_w2, gm_b2, gm_w3, gm_b3, True, False)

    w1a, w1b = _build_pair_taps(gw_w1)
    w2a, w2b = _build_pair_taps(gw_w2)
    pooled = pl.pallas_call(
        _gw_kernel,
        grid=(nb,),
        in_specs=[
            pl.BlockSpec((1, 18, 18, _C1), lambda b: (b, 0, 0, 0)),
            _full((9 * _C1, 2 * _C1)), _full((6 * _C1, 2 * _C1)),
            _full((1, 2 * _C1)),
            _full((9 * _C1, 2 * _CK)), _full((6 * _C1, 2 * _CK)),
            _full((1, _CK)),
        ],
        out_specs=pl.BlockSpec((1, 1, _CK), lambda b: (b, 0, 0)),
        out_shape=jax.ShapeDtypeStruct((nb, 1, _CK), jnp.float32),
        scratch_shapes=[pltpu.VMEM((34, 34, _C1), bf)],
        compiler_params=pltpu.CompilerParams(
            dimension_semantics=("arbitrary",),
            vmem_limit_bytes=_VMEM_LIMIT,
        ),
        name="gmm_gw_pool",
    )(z1p, w1a.astype(bf), w1b.astype(bf), dbl(gw_b1),
      w2a.astype(bf), w2b.astype(bf), gw_b2.reshape(1, _CK))

    weights = pl.pallas_call(
        _tail_kernel,
        in_specs=[
            pl.BlockSpec((nb, 1, _CK), lambda: (0, 0, 0)),
            pl.BlockSpec((_CK, _CK), lambda: (0, 0)),
            pl.BlockSpec((1, _CK), lambda: (0, 0)),
        ],
        out_specs=pl.BlockSpec((nb, _CK), lambda: (0, 0)),
        out_shape=jax.ShapeDtypeStruct((nb, _CK), jnp.float32),
        compiler_params=pltpu.CompilerParams(
            vmem_limit_bytes=_VMEM_LIMIT,
        ),
        name="gmm_gw_tail",
    )(pooled, gw_w3[:, :, 0, 0].transpose(1, 0), gw_b3.reshape(1, _CK))

    return sigma, means, weights.reshape(nb, _CK, 1, 1)


def kernel(z1, gs_w1, gs_b1, gs_w2, gs_b2, gs_w3, gs_b3,
           gm_w1, gm_b1, gm_w2, gm_b2, gm_w3, gm_b3,
           gw_w1, gw_b1, gw_w2, gw_b2, gw_w3, gw_b3):
    args = (z1, gs_w1, gs_b1, gs_w2, gs_b2, gs_w3, gs_b3,
            gm_w1, gm_b1, gm_w2, gm_b2, gm_w3, gm_b3,
            gw_w1, gw_b1, gw_w2, gw_b2, gw_w3, gw_b3)
    devs = jax.devices()
    nd = 2 if len(devs) >= 2 and z1.shape[0] % 2 == 0 else 1
    if nd == 1:
        return _pipeline(*args)
    mesh = Mesh(np.array(devs[:nd]), ("d",))
    in_specs = (P("d"),) + (P(),) * 18
    return jax.shard_map(_pipeline, mesh=mesh, in_specs=in_specs,
                         out_specs=P("d"), check_vma=False)(*args)
